# Initial kernel scaffold; baseline (speedup 1.0000x reference)
import jax
import jax.numpy as jnp
from jax.experimental import pallas as pl


def kernel(x, edge_index, W1l, b1l, W1r, W2l, b2l, W2r, W3l, b3l, W3r, Wfc1, bfc1, ln_g, ln_b, prelu_a, Wfc2, bfc2):
    raise NotImplementedError("write your pallas kernel here")


# speedup vs baseline: 60.6348x; 60.6348x over previous
import functools

import jax
import jax.numpy as jnp
from jax import lax
from jax.experimental import pallas as pl
from jax.experimental.pallas import tpu as pltpu
from jax.experimental.pallas import tpu_sc as plsc

N = 10000
E = 320000
D = 32
LO = 40
HI = 142
R = HI - LO
ACCW = 48
AW = 104
ASZ = R * AW + 16
NC = 2
NS = 16
NW = NC * NS
EPT = E // NW
G = 128
CODES = EPT + G


def _sc_body(x_hbm, ei_hbm, pacc_hbm, pa_hbm,
             src_v, dst_v, codes_v, cntv_v, idx_v, rows_v, acc_v, acca_v,
             sem):
    wid = lax.axis_index("s") * NC + lax.axis_index("c")
    ebase = wid * EPT

    pltpu.sync_copy(ei_hbm.at[pl.ds(ebase, EPT)], src_v)
    pltpu.sync_copy(ei_hbm.at[pl.ds(E + ebase, EPT)], dst_v)

    z16i = jnp.zeros((16,), jnp.int32)
    z16f = jnp.zeros((16,), jnp.float32)
    one16i = jnp.ones((16,), jnp.int32)

    def zero_i(i, _):
        codes_v[pl.ds(i * 16, 16)] = z16i
        return 0

    lax.fori_loop(0, CODES // 16, zero_i, 0)

    def zero_acc(i, _):
        acc_v[pl.ds(i * 16, 16)] = z16f
        return 0

    lax.fori_loop(0, (R * ACCW) // 16, zero_acc, 0)

    def zero_a(i, _):
        acca_v[pl.ds(i * 16, 16)] = z16f
        return 0

    lax.fori_loop(0, ASZ // 16, zero_a, 0)

    cntv_v[pl.ds(0, 16)] = z16i

    def compact(i, _):
        s = src_v[pl.ds(i * 16, 16)]
        dd = dst_v[pl.ds(i * 16, 16)]
        m = (dd >= LO) & (dd < HI)
        code = lax.shift_left(s, 7) + (dd - LO)
        mi = jnp.where(m, one16i, z16i)
        tv = cntv_v[pl.ds(0, 16)]
        pos = jnp.maximum(tv + plsc.cumsum(mi) - one16i, z16i)
        plsc.store_scatter(codes_v, [pos], code, mask=m)
        cntv_v[pl.ds(0, 16)] = tv + plsc.all_reduce_population_count(m)
        return 0

    lax.fori_loop(0, EPT // 16, compact, 0)
    cnt = cntv_v[pl.ds(0, 16)][0]

    e0 = jnp.where(lax.iota(jnp.int32, 16) == 0, 1.0, 0.0)
    lanes = lax.iota(jnp.int32, 16)

    def chunk(k, _):
        base = k * G
        for j in range(G // 16):
            idx_v[pl.ds(j * 16, 16)] = lax.shift_right_logical(
                codes_v[pl.ds(base + j * 16, 16)], 7)
        pltpu.async_copy(x_hbm.at[idx_v], rows_v, sem).wait()
        nthis = jnp.minimum(G, cnt - base)

        def edge(j, _):
            code = codes_v[pl.ds(base + j, 16)][0]
            dloc = code & 127
            s = lax.shift_right_logical(code, 7)
            rb = dloc * ACCW
            acc_v[pl.ds(rb, 16)] = acc_v[pl.ds(rb, 16)] + rows_v[j, pl.ds(0, 16)]
            acc_v[pl.ds(rb + 16, 16)] = (
                acc_v[pl.ds(rb + 16, 16)] + rows_v[j, pl.ds(16, 16)])
            acc_v[pl.ds(rb + 32, 16)] = acc_v[pl.ds(rb + 32, 16)] + e0
            sl = jnp.clip(s - LO, 0, R - 1)
            q = lax.shift_right_logical(sl, 4)
            lane = sl & 15
            f = jnp.where((s >= LO) & (s < HI), 1.0, 0.0)
            oh = jnp.where(lanes == lane, f, 0.0)
            pa = dloc * AW + lax.shift_left(q, 4)
            acca_v[pl.ds(pa, 16)] = acca_v[pl.ds(pa, 16)] + oh
            return 0

        lax.fori_loop(0, nthis, edge, 0)
        return 0

    nch = (cnt + G - 1) // G
    lax.fori_loop(0, nch, chunk, 0)

    pltpu.sync_copy(acc_v, pacc_hbm.at[wid])
    pltpu.sync_copy(acca_v, pa_hbm.at[wid])


_sc_pass = functools.partial(
    pl.kernel,
    out_type=(jax.ShapeDtypeStruct((NW, R * ACCW), jnp.float32),
              jax.ShapeDtypeStruct((NW, ASZ), jnp.float32)),
    mesh=plsc.VectorSubcoreMesh(core_axis_name="c", subcore_axis_name="s",
                                num_cores=NC, num_subcores=NS),
    scratch_types=[
        pltpu.VMEM((EPT,), jnp.int32),
        pltpu.VMEM((EPT,), jnp.int32),
        pltpu.VMEM((CODES,), jnp.int32),
        pltpu.VMEM((16,), jnp.int32),
        pltpu.VMEM((G,), jnp.int32),
        pltpu.VMEM((G, D), jnp.float32),
        pltpu.VMEM((R * ACCW,), jnp.float32),
        pltpu.VMEM((ASZ,), jnp.float32),
        pltpu.SemaphoreType.DMA,
    ],
    compiler_params=pltpu.CompilerParams(
        needs_layout_passes=False,
        use_tc_tiling_on_sc=False,
    ),
)(_sc_body)


def _tc1_body(pacc_ref, pa_ref, x_ref,
              w1l_ref, w1r_ref, w2l_ref, w2r_ref, w3l_ref, w3r_ref,
              b1_ref, b2_ref, b3_ref, out_ref):
    s = jnp.sum(pacc_ref[...], axis=0)
    t = s[:, :D]
    cm = jnp.maximum(s[:, D:D + 1], 1.0)
    a = jnp.sum(pa_ref[...], axis=0)
    x = x_ref[...]

    def dot(u, v):
        return jnp.dot(u, v, preferred_element_type=jnp.float32)

    agg1 = t[0:44] / cm[0:44]
    h1 = jnp.maximum(dot(agg1, w1l_ref[...]) + b1_ref[...]
                     + dot(x[40:84], w1r_ref[...]), 0.0)
    d1 = h1 - x[40:84]

    sum2 = t[44:72] + dot(a[44:72, 0:44], d1)
    h2 = jnp.maximum(dot(sum2 / cm[44:72], w2l_ref[...]) + b2_ref[...]
                     + dot(x[84:112], w2r_ref[...]), 0.0)
    d2 = h2 - x[84:112]

    sum3 = (t[72:102] + dot(a[72:102, 0:44], d1)
            + dot(a[72:102, 44:72], d2))
    h3 = jnp.maximum(dot(sum3 / cm[72:102], w3l_ref[...]) + b3_ref[...]
                     + dot(x[112:142], w3r_ref[...]), 0.0)

    out_ref[...] = jnp.concatenate(
        [x[1:8], x[17:22], x[29:32], x[37:38],
         h1[1:12], h1[25:34], h2[1:14], h3[0:30]], axis=0)


def _tc2_body(feat_ref, wfc1_ref, bfc1_ref, g_ref, b_ref, pa_ref,
              wfc2_ref, bfc2_ref, out_ref):
    h = (jnp.dot(feat_ref[...], wfc1_ref[...],
                 preferred_element_type=jnp.float32) + bfc1_ref[...])
    mu = jnp.mean(h)
    var = jnp.mean((h - mu) ** 2)
    h = (h - mu) / jnp.sqrt(var + 1e-5) * g_ref[...] + b_ref[...]
    h = jnp.where(h >= 0.0, h, pa_ref[0, 0] * h)
    o = (jnp.dot(h, wfc2_ref[...], preferred_element_type=jnp.float32)
         + bfc2_ref[...])
    out_ref[...] = jnp.log1p(jnp.exp(-jnp.abs(o))) + jnp.maximum(o, 0.0)


def kernel(x, edge_index, W1l, b1l, W1r, W2l, b2l, W2r, W3l, b3l, W3r,
           Wfc1, bfc1, ln_g, ln_b, prelu_a, Wfc2, bfc2):
    pacc, pa = _sc_pass(x, edge_index.reshape(2 * E))

    pacc3 = pacc.reshape(NW, R, ACCW)
    pa3 = pa[:, :R * AW].reshape(NW, R, AW)[:, :, :R]

    feat = pl.pallas_call(
        _tc1_body,
        out_shape=jax.ShapeDtypeStruct((79, D), jnp.float32),
    )(pacc3, pa3, x[:HI],
      W1l.T, W1r.T, W2l.T, W2r.T, W3l.T, W3r.T,
      b1l.reshape(1, D), b2l.reshape(1, D), b3l.reshape(1, D))

    out = pl.pallas_call(
        _tc2_body,
        out_shape=jax.ShapeDtypeStruct((1, 5), jnp.float32),
    )(feat.reshape(1, 79 * D), Wfc1.T, bfc1.reshape(1, 256),
      ln_g.reshape(1, 256), ln_b.reshape(1, 256),
      prelu_a.reshape(1, 1), Wfc2.T, bfc2.reshape(1, 5))

    return out.reshape(5)
